# Initial kernel scaffold; baseline (speedup 1.0000x reference)
import jax
import jax.numpy as jnp
from jax.experimental import pallas as pl


def kernel(x, edge_index, W1, b1, W2, b2, Wfc, bfc):
    raise NotImplementedError("write your pallas kernel here")


# speedup vs baseline: 22.3448x; 22.3448x over previous
import functools

import jax
import jax.numpy as jnp
from jax import lax
from jax.experimental import pallas as pl
from jax.experimental.pallas import tpu as pltpu
from jax.experimental.pallas import tpu_sc as plsc

N = 10000
NPAD = 10240
E = 320000
EPAD = 327680
D = 128
A = 16
NC, NS = 2, 16
NW = NC * NS
CHUNK = 128
CHUNKS = EPAD // (NW * CHUNK)
ROWS_PT = NPAD // NS

_mesh = plsc.VectorSubcoreMesh(
    core_axis_name="c", subcore_axis_name="s", num_cores=NC, num_subcores=NS)


@functools.partial(
    pl.kernel,
    out_type=jax.ShapeDtypeStruct((NC, NS, ROWS_PT), jnp.float32),
    mesh=_mesh,
    scratch_types=[
        pltpu.VMEM((CHUNKS, CHUNK), jnp.int32),
        pltpu.VMEM((CHUNK,), jnp.float32),
        pltpu.VMEM_SHARED((NPAD,), jnp.float32),
    ],
)
def _deg_kernel(dst_hbm, zeros_hbm, out_hbm, dst_v, ones_v, deg_sh):
    cid = lax.axis_index("c")
    sid = lax.axis_index("s")
    wid = cid * NS + sid
    pltpu.sync_copy(zeros_hbm.at[pl.ds(sid * ROWS_PT, ROWS_PT)],
                    deg_sh.at[pl.ds(sid * ROWS_PT, ROWS_PT)])
    pltpu.sync_copy(dst_hbm.at[pl.ds(wid * CHUNKS, CHUNKS)], dst_v)
    for i in range(CHUNK // 16):
        ones_v[pl.ds(i * 16, 16)] = jnp.full((16,), 1.0, jnp.float32)
    plsc.subcore_barrier()

    def body(j, carry):
        pltpu.sync_copy(ones_v, deg_sh.at[dst_v.at[j]], add=True)
        return carry

    lax.fori_loop(0, CHUNKS, body, 0)
    plsc.subcore_barrier()
    pltpu.sync_copy(deg_sh.at[pl.ds(sid * ROWS_PT, ROWS_PT)],
                    out_hbm.at[cid, sid])


@functools.partial(
    pl.kernel,
    out_type=jax.ShapeDtypeStruct((NC, NS, ROWS_PT, D), jnp.float32),
    mesh=_mesh,
    scratch_types=[
        pltpu.VMEM((CHUNKS, CHUNK), jnp.int32),
        pltpu.VMEM((CHUNKS, CHUNK), jnp.int32),
        pltpu.VMEM((CHUNK, D), jnp.float32),
        pltpu.VMEM_SHARED((NPAD, D), jnp.float32),
        pltpu.SemaphoreType.DMA,
    ],
)
def _agg_kernel(hs_hbm, src_hbm, dst_hbm, zeros_hbm, out_hbm,
                src_v, dst_v, rows0, acc_sh, sem):
    cid = lax.axis_index("c")
    sid = lax.axis_index("s")
    wid = cid * NS + sid
    pltpu.sync_copy(zeros_hbm.at[pl.ds(sid * ROWS_PT, ROWS_PT)],
                    acc_sh.at[pl.ds(sid * ROWS_PT, ROWS_PT)])
    pltpu.sync_copy(src_hbm.at[pl.ds(wid * CHUNKS, CHUNKS)], src_v)
    pltpu.sync_copy(dst_hbm.at[pl.ds(wid * CHUNKS, CHUNKS)], dst_v)
    plsc.subcore_barrier()

    def body(j, carry):
        pltpu.async_copy(hs_hbm.at[src_v.at[j]], rows0, sem)
        pltpu.make_async_copy(hs_hbm.at[src_v.at[j]], rows0, sem).wait()
        pltpu.sync_copy(rows0, acc_sh.at[dst_v.at[j]], add=True)
        return carry

    lax.fori_loop(0, CHUNKS, body, 0)
    plsc.subcore_barrier()
    pltpu.sync_copy(acc_sh.at[pl.ds(sid * ROWS_PT, ROWS_PT)],
                    out_hbm.at[cid, sid])


_RB = 1280
_GRID = NPAD // _RB


def _tc_pre_body(degT_ref, x_ref, w_ref, hs_ref, dinv_ref):
    deg = degT_ref[:, 0:1] + degT_ref[:, 1:2] + 1.0
    dinv = lax.rsqrt(deg)
    h = jnp.dot(x_ref[...], w_ref[...], preferred_element_type=jnp.float32)
    hs_ref[...] = h * dinv
    dinv_ref[...] = dinv


def _tc_pre(degT, x, w):
    return pl.pallas_call(
        _tc_pre_body,
        grid=(_GRID,),
        in_specs=[
            pl.BlockSpec((_RB, 2), lambda i: (i, 0)),
            pl.BlockSpec((_RB, D), lambda i: (i, 0)),
            pl.BlockSpec((D, D), lambda i: (0, 0)),
        ],
        out_specs=[
            pl.BlockSpec((_RB, D), lambda i: (i, 0)),
            pl.BlockSpec((_RB, 1), lambda i: (i, 0)),
        ],
        out_shape=[
            jax.ShapeDtypeStruct((NPAD, D), jnp.float32),
            jax.ShapeDtypeStruct((NPAD, 1), jnp.float32),
        ],
    )(degT, x, w)


def _tc_mid_body(p_ref, hs_ref, dinv_ref, b_ref, w_ref, out_ref):
    t = (p_ref[0] + p_ref[1] + hs_ref[...]) * dinv_ref[...] + b_ref[...]
    o = jnp.maximum(t, 0.0)
    out_ref[...] = jnp.dot(
        o, w_ref[...], preferred_element_type=jnp.float32) * dinv_ref[...]


def _tc_mid(p, hs, dinv, b, w):
    return pl.pallas_call(
        _tc_mid_body,
        grid=(_GRID,),
        in_specs=[
            pl.BlockSpec((NC, _RB, D), lambda i: (0, i, 0)),
            pl.BlockSpec((_RB, D), lambda i: (i, 0)),
            pl.BlockSpec((_RB, 1), lambda i: (i, 0)),
            pl.BlockSpec((1, D), lambda i: (0, 0)),
            pl.BlockSpec((D, D), lambda i: (0, 0)),
        ],
        out_specs=pl.BlockSpec((_RB, D), lambda i: (i, 0)),
        out_shape=jax.ShapeDtypeStruct((NPAD, D), jnp.float32),
    )(p, hs, dinv, b, w)


def _tc_head_body(p_ref, hs_ref, dinv_ref, b_ref, wfc_ref, bfc_ref, out_ref):
    t = (p_ref[0] + p_ref[1] + hs_ref[...]) * dinv_ref[...] + b_ref[...]
    o = jnp.maximum(t, 0.0)
    out_ref[...] = jnp.dot(
        o, wfc_ref[...], preferred_element_type=jnp.float32) + bfc_ref[...]


def _tc_head(p, hs, dinv, b, wfc, bfc):
    return pl.pallas_call(
        _tc_head_body,
        grid=(_GRID,),
        in_specs=[
            pl.BlockSpec((NC, _RB, D), lambda i: (0, i, 0)),
            pl.BlockSpec((_RB, D), lambda i: (i, 0)),
            pl.BlockSpec((_RB, 1), lambda i: (i, 0)),
            pl.BlockSpec((1, D), lambda i: (0, 0)),
            pl.BlockSpec((D, A), lambda i: (0, 0)),
            pl.BlockSpec((1, A), lambda i: (0, 0)),
        ],
        out_specs=pl.BlockSpec((_RB, A), lambda i: (i, 0)),
        out_shape=jax.ShapeDtypeStruct((NPAD, A), jnp.float32),
    )(p, hs, dinv, b, wfc, bfc)


def kernel(x, edge_index, W1, b1, W2, b2, Wfc, bfc):
    ei = edge_index.astype(jnp.int32)
    npad = EPAD - E
    pad_src = (jnp.arange(npad, dtype=jnp.int32) * 7919) % N
    pad_dst = N + jnp.arange(npad, dtype=jnp.int32) % (NPAD - N)
    src2d = jnp.concatenate([ei[0], pad_src]).reshape(EPAD // CHUNK, CHUNK)
    dst2d = jnp.concatenate([ei[1], pad_dst]).reshape(EPAD // CHUNK, CHUNK)
    zeros1 = jnp.zeros((NPAD,), jnp.float32)
    zeros2 = jnp.zeros((NPAD, D), jnp.float32)
    x_pad = jnp.concatenate([x, jnp.zeros((NPAD - N, D), jnp.float32)])

    degp = _deg_kernel(dst2d, zeros1)
    degT = degp.reshape(NC, NPAD).T
    hs1, dinv = _tc_pre(degT, x_pad, W1)
    p1 = _agg_kernel(hs1, src2d, dst2d, zeros2).reshape(NC, NPAD, D)
    hs2 = _tc_mid(p1, hs1, dinv, b1.reshape(1, D), W2)
    p2 = _agg_kernel(hs2, src2d, dst2d, zeros2).reshape(NC, NPAD, D)
    q = _tc_head(p2, hs2, dinv, b2.reshape(1, D), Wfc, bfc.reshape(1, A))
    return q[:N]


# speedup vs baseline: 32.6011x; 1.4590x over previous
import functools

import jax
import jax.numpy as jnp
from jax import lax
from jax.experimental import pallas as pl
from jax.experimental.pallas import tpu as pltpu
from jax.experimental.pallas import tpu_sc as plsc

N = 10000
NPAD = 10240
E = 320000
EPAD = 327680
D = 128
A = 16
NC, NS = 2, 16
NW = NC * NS
CHUNK = 128
CHUNKS = EPAD // (NW * CHUNK)
ROWS_PT = NPAD // NS

_mesh = plsc.VectorSubcoreMesh(
    core_axis_name="c", subcore_axis_name="s", num_cores=NC, num_subcores=NS)


@functools.partial(
    pl.kernel,
    out_type=jax.ShapeDtypeStruct((NC, NS, ROWS_PT), jnp.float32),
    mesh=_mesh,
    scratch_types=[
        pltpu.VMEM((CHUNKS, CHUNK), jnp.int32),
        pltpu.VMEM((CHUNK,), jnp.float32),
        pltpu.VMEM_SHARED((NPAD,), jnp.float32),
    ],
)
def _deg_kernel(dst_hbm, zeros_hbm, out_hbm, dst_v, ones_v, deg_sh):
    cid = lax.axis_index("c")
    sid = lax.axis_index("s")
    wid = cid * NS + sid
    pltpu.sync_copy(zeros_hbm.at[pl.ds(sid * ROWS_PT, ROWS_PT)],
                    deg_sh.at[pl.ds(sid * ROWS_PT, ROWS_PT)])
    pltpu.sync_copy(dst_hbm.at[pl.ds(wid * CHUNKS, CHUNKS)], dst_v)
    for i in range(CHUNK // 16):
        ones_v[pl.ds(i * 16, 16)] = jnp.full((16,), 1.0, jnp.float32)
    plsc.subcore_barrier()

    def body(j, carry):
        pltpu.sync_copy(ones_v, deg_sh.at[dst_v.at[j]], add=True)
        return carry

    lax.fori_loop(0, CHUNKS, body, 0)
    plsc.subcore_barrier()
    pltpu.sync_copy(deg_sh.at[pl.ds(sid * ROWS_PT, ROWS_PT)],
                    out_hbm.at[cid, sid])


@functools.partial(
    pl.kernel,
    out_type=jax.ShapeDtypeStruct((NC, NS, ROWS_PT, D), jnp.float32),
    mesh=_mesh,
    scratch_types=[
        pltpu.VMEM((CHUNKS, CHUNK), jnp.int32),
        pltpu.VMEM((CHUNK,), jnp.int32),
        pltpu.VMEM((CHUNK,), jnp.int32),
        pltpu.VMEM((CHUNK,), jnp.int32),
        pltpu.VMEM((CHUNK,), jnp.int32),
        pltpu.VMEM((CHUNK, D), jnp.float32),
        pltpu.VMEM((CHUNK, D), jnp.float32),
        pltpu.VMEM_SHARED((NPAD, D), jnp.float32),
        pltpu.SemaphoreType.DMA,
        pltpu.SemaphoreType.DMA,
    ],
)
def _agg_kernel(hs_hbm, packed_hbm, zeros_hbm, out_hbm,
                packed_v, src0, src1, dst0, dst1, rows0, rows1, acc_sh,
                sem0, sem1):
    cid = lax.axis_index("c")
    sid = lax.axis_index("s")
    wid = cid * NS + sid
    pltpu.sync_copy(zeros_hbm.at[pl.ds(sid * ROWS_PT, ROWS_PT)],
                    acc_sh.at[pl.ds(sid * ROWS_PT, ROWS_PT)])
    pltpu.sync_copy(packed_hbm.at[pl.ds(wid * CHUNKS, CHUNKS)], packed_v)
    plsc.subcore_barrier()

    def unpack(j, src_buf, dst_buf):
        for k in range(CHUNK // 16):
            xv = packed_v[j, pl.ds(k * 16, 16)]
            src_buf[pl.ds(k * 16, 16)] = jnp.bitwise_and(xv, 16383)
            dst_buf[pl.ds(k * 16, 16)] = jnp.right_shift(xv, 14)

    unpack(0, src0, dst0)
    unpack(1, src1, dst1)
    pltpu.async_copy(hs_hbm.at[src0], rows0, sem0)
    pltpu.async_copy(hs_hbm.at[src1], rows1, sem1)

    def body(i, carry):
        j0 = 2 * i
        pltpu.make_async_copy(hs_hbm.at[src0], rows0, sem0).wait()
        pltpu.sync_copy(rows0, acc_sh.at[dst0], add=True)
        unpack(j0 + 2, src0, dst0)
        pltpu.async_copy(hs_hbm.at[src0], rows0, sem0)
        pltpu.make_async_copy(hs_hbm.at[src1], rows1, sem1).wait()
        pltpu.sync_copy(rows1, acc_sh.at[dst1], add=True)
        unpack(j0 + 3, src1, dst1)
        pltpu.async_copy(hs_hbm.at[src1], rows1, sem1)
        return carry

    lax.fori_loop(0, CHUNKS // 2 - 1, body, 0)
    pltpu.make_async_copy(hs_hbm.at[src0], rows0, sem0).wait()
    pltpu.sync_copy(rows0, acc_sh.at[dst0], add=True)
    pltpu.make_async_copy(hs_hbm.at[src1], rows1, sem1).wait()
    pltpu.sync_copy(rows1, acc_sh.at[dst1], add=True)
    plsc.subcore_barrier()
    pltpu.sync_copy(acc_sh.at[pl.ds(sid * ROWS_PT, ROWS_PT)],
                    out_hbm.at[cid, sid])


_RB = 1280
_GRID = NPAD // _RB


def _tc_pre_body(degT_ref, x_ref, w_ref, hs_ref, dinv_ref):
    deg = degT_ref[:, 0:1] + degT_ref[:, 1:2] + 1.0
    dinv = lax.rsqrt(deg)
    h = jnp.dot(x_ref[...], w_ref[...], preferred_element_type=jnp.float32)
    hs_ref[...] = h * dinv
    dinv_ref[...] = dinv


def _tc_pre(degT, x, w):
    return pl.pallas_call(
        _tc_pre_body,
        grid=(_GRID,),
        in_specs=[
            pl.BlockSpec((_RB, 2), lambda i: (i, 0)),
            pl.BlockSpec((_RB, D), lambda i: (i, 0)),
            pl.BlockSpec((D, D), lambda i: (0, 0)),
        ],
        out_specs=[
            pl.BlockSpec((_RB, D), lambda i: (i, 0)),
            pl.BlockSpec((_RB, 1), lambda i: (i, 0)),
        ],
        out_shape=[
            jax.ShapeDtypeStruct((NPAD, D), jnp.float32),
            jax.ShapeDtypeStruct((NPAD, 1), jnp.float32),
        ],
    )(degT, x, w)


def _tc_mid_body(p_ref, hs_ref, dinv_ref, b_ref, w_ref, out_ref):
    t = (p_ref[0] + p_ref[1] + hs_ref[...]) * dinv_ref[...] + b_ref[...]
    o = jnp.maximum(t, 0.0)
    out_ref[...] = jnp.dot(
        o, w_ref[...], preferred_element_type=jnp.float32) * dinv_ref[...]


def _tc_mid(p, hs, dinv, b, w):
    return pl.pallas_call(
        _tc_mid_body,
        grid=(_GRID,),
        in_specs=[
            pl.BlockSpec((NC, _RB, D), lambda i: (0, i, 0)),
            pl.BlockSpec((_RB, D), lambda i: (i, 0)),
            pl.BlockSpec((_RB, 1), lambda i: (i, 0)),
            pl.BlockSpec((1, D), lambda i: (0, 0)),
            pl.BlockSpec((D, D), lambda i: (0, 0)),
        ],
        out_specs=pl.BlockSpec((_RB, D), lambda i: (i, 0)),
        out_shape=jax.ShapeDtypeStruct((NPAD, D), jnp.float32),
    )(p, hs, dinv, b, w)


def _tc_head_body(p_ref, hs_ref, dinv_ref, b_ref, wfc_ref, bfc_ref, out_ref):
    t = (p_ref[0] + p_ref[1] + hs_ref[...]) * dinv_ref[...] + b_ref[...]
    o = jnp.maximum(t, 0.0)
    out_ref[...] = jnp.dot(
        o, wfc_ref[...], preferred_element_type=jnp.float32) + bfc_ref[...]


def _tc_head(p, hs, dinv, b, wfc, bfc):
    return pl.pallas_call(
        _tc_head_body,
        grid=(_GRID,),
        in_specs=[
            pl.BlockSpec((NC, _RB, D), lambda i: (0, i, 0)),
            pl.BlockSpec((_RB, D), lambda i: (i, 0)),
            pl.BlockSpec((_RB, 1), lambda i: (i, 0)),
            pl.BlockSpec((1, D), lambda i: (0, 0)),
            pl.BlockSpec((D, A), lambda i: (0, 0)),
            pl.BlockSpec((1, A), lambda i: (0, 0)),
        ],
        out_specs=pl.BlockSpec((_RB, A), lambda i: (i, 0)),
        out_shape=jax.ShapeDtypeStruct((NPAD, A), jnp.float32),
    )(p, hs, dinv, b, wfc, bfc)


def kernel(x, edge_index, W1, b1, W2, b2, Wfc, bfc):
    ei = edge_index.astype(jnp.int32)
    npad = EPAD - E
    pad_src = (jnp.arange(npad, dtype=jnp.int32) * 7919) % N
    pad_dst = N + jnp.arange(npad, dtype=jnp.int32) % (NPAD - N)
    src2d = jnp.concatenate([ei[0], pad_src]).reshape(EPAD // CHUNK, CHUNK)
    dst2d = jnp.concatenate([ei[1], pad_dst]).reshape(EPAD // CHUNK, CHUNK)
    packed2d = (dst2d << 14) | src2d
    zeros1 = jnp.zeros((NPAD,), jnp.float32)
    zeros2 = jnp.zeros((NPAD, D), jnp.float32)
    x_pad = jnp.concatenate([x, jnp.zeros((NPAD - N, D), jnp.float32)])

    degp = _deg_kernel(dst2d, zeros1)
    degT = degp.reshape(NC, NPAD).T
    hs1, dinv = _tc_pre(degT, x_pad, W1)
    p1 = _agg_kernel(hs1, packed2d, zeros2).reshape(NC, NPAD, D)
    hs2 = _tc_mid(p1, hs1, dinv, b1.reshape(1, D), W2)
    p2 = _agg_kernel(hs2, packed2d, zeros2).reshape(NC, NPAD, D)
    q = _tc_head(p2, hs2, dinv, b2.reshape(1, D), Wfc, bfc.reshape(1, A))
    return q[:N]
